# baseline (device time: 24792 ns/iter reference)
import jax
import jax.numpy as jnp
from jax import lax
from jax.experimental import pallas as pl
from jax.experimental.pallas import tpu as pltpu


def kernel(x, router, W1, W2):
    t_loc, d = x.shape
    e_loc, _, f = W1.shape

    def body(x_ref, r_ref, w1_ref, w2_ref, out_ref,
             xsend, xrec, rrec, wsend, wrec, ysend, yrec,
             send_sems, recv_sems):
        my_x = lax.axis_index("x")
        my_y = lax.axis_index("y")
        my_z = lax.axis_index("z")
        partner = (my_x, my_y, 1 - my_z)
        z0 = my_z == 0

        def rdma(src, dst, i):
            return pltpu.make_async_remote_copy(
                src_ref=src, dst_ref=dst,
                send_sem=send_sems.at[i], recv_sem=recv_sems.at[i],
                device_id=partner, device_id_type=pl.DeviceIdType.MESH,
            )

        barrier = pltpu.get_barrier_semaphore()
        pl.semaphore_signal(barrier, inc=1, device_id=partner,
                            device_id_type=pl.DeviceIdType.MESH)
        pl.semaphore_wait(barrier, 1)

        rdma_r = rdma(r_ref, rrec, 0)
        rdma_r.start()
        xsend[...] = x_ref[...].astype(jnp.bfloat16)
        rdma_x = rdma(xsend, xrec, 1)
        rdma_x.start()

        rdma_r.wait()
        xf = x_ref[...]
        gm = jnp.dot(xf, r_ref[...], preferred_element_type=jnp.float32)
        go = jnp.dot(xf, rrec[...], preferred_element_type=jnp.float32)
        cols = [
            jnp.where(z0, gm[:, 0:1], go[:, 0:1]),
            jnp.where(z0, gm[:, 1:2], go[:, 1:2]),
            jnp.where(z0, go[:, 0:1], gm[:, 0:1]),
            jnp.where(z0, go[:, 1:2], gm[:, 1:2]),
        ]
        m = jnp.maximum(jnp.maximum(cols[0], cols[1]),
                        jnp.maximum(cols[2], cols[3]))
        w = []
        for e in range(4):
            rank = sum(
                jnp.where(cols[o] >= cols[e] if o < e else cols[o] > cols[e],
                          1, 0)
                for o in range(4) if o != e)
            w.append(jnp.where(rank < 2, jnp.exp(cols[e] - m), 0.0))
        denom = w[0] + w[1] + w[2] + w[3]
        w = [wi / denom for wi in w]
        wmine = [jnp.where(z0, w[j], w[2 + j]) for j in range(e_loc)]
        wsend[...] = jnp.concatenate(
            [jnp.where(z0, w[2 + j], w[j]) for j in range(e_loc)], axis=1)
        rdma_w = rdma(wsend, wrec, 2)
        rdma_w.start()

        xl = xsend[...]
        acc_loc = jnp.zeros((t_loc, d), jnp.float32)
        for j in range(e_loc):
            w1 = w1_ref[j].astype(jnp.bfloat16)
            w2 = w2_ref[j].astype(jnp.bfloat16)
            h = jnp.dot(xl, w1, preferred_element_type=jnp.float32)
            h = jnp.maximum(h, 0.0).astype(jnp.bfloat16)
            acc_loc += jnp.dot(h, w2, preferred_element_type=jnp.float32) \
                * wmine[j]

        rdma_x.wait()
        rdma_w.wait()
        xr = xrec[...]
        acc_rec = jnp.zeros((t_loc, d), jnp.float32)
        for j in range(e_loc):
            w1 = w1_ref[j].astype(jnp.bfloat16)
            w2 = w2_ref[j].astype(jnp.bfloat16)
            h = jnp.dot(xr, w1, preferred_element_type=jnp.float32)
            h = jnp.maximum(h, 0.0).astype(jnp.bfloat16)
            acc_rec += jnp.dot(h, w2, preferred_element_type=jnp.float32) \
                * wrec[:, j:j + 1]

        ysend[...] = acc_rec.astype(jnp.bfloat16)
        rdma_y = rdma(ysend, yrec, 3)
        rdma_y.start()
        rdma_y.wait()
        out_ref[...] = acc_loc + yrec[...].astype(jnp.float32)

    return pl.pallas_call(
        body,
        out_shape=jax.ShapeDtypeStruct((t_loc, d), jnp.float32),
        in_specs=[pl.BlockSpec(memory_space=pltpu.VMEM)] * 4,
        out_specs=pl.BlockSpec(memory_space=pltpu.VMEM),
        scratch_shapes=[
            pltpu.VMEM((t_loc, d), jnp.bfloat16),
            pltpu.VMEM((t_loc, d), jnp.bfloat16),
            pltpu.VMEM((d, e_loc), jnp.float32),
            pltpu.VMEM((t_loc, e_loc), jnp.float32),
            pltpu.VMEM((t_loc, e_loc), jnp.float32),
            pltpu.VMEM((t_loc, d), jnp.bfloat16),
            pltpu.VMEM((t_loc, d), jnp.bfloat16),
            pltpu.SemaphoreType.DMA((4,)),
            pltpu.SemaphoreType.DMA((4,)),
        ],
        compiler_params=pltpu.CompilerParams(collective_id=0),
    )(x, router, W1, W2)
